# baseline (speedup 1.0000x reference)
import functools

import jax
import jax.numpy as jnp
from jax import lax
from jax.experimental import pallas as pl
from jax.experimental.pallas import tpu as pltpu
from jax.experimental.pallas import tpu_sc as plsc


def kernel(row_embed, col_embed):
    H, D2 = row_embed.shape
    W = col_embed.shape[0]

    NW = 32
    RPW = H // NW
    L = 16

    mesh = plsc.VectorSubcoreMesh(core_axis_name="c", subcore_axis_name="s")

    @functools.partial(
        pl.kernel,
        mesh=mesh,
        out_type=jax.ShapeDtypeStruct((H * W, 2 * D2), jnp.float32),
        scratch_types=[
            pltpu.VMEM((RPW, W), jnp.int32),
            pltpu.VMEM((W, D2), jnp.float32),
            pltpu.VMEM((W, D2), jnp.float32),
            pltpu.VMEM((W, D2), jnp.float32),
            pltpu.SemaphoreType.DMA,
            pltpu.SemaphoreType.DMA,
            pltpu.SemaphoreType.DMA,
            pltpu.SemaphoreType.DMA,
        ],
    )
    def dpos_kernel(row_hbm, col_hbm, out_hbm, idx_v, col_v, r0, r1, csem, g0, g1, wsem):
        wid = lax.axis_index("s") * 2 + lax.axis_index("c")
        base_i = wid * RPW
        col_read = pltpu.async_copy(col_hbm, col_v, csem)
        for t in range(RPW):
            rep = jnp.broadcast_to(base_i + t, (L,)).astype(jnp.int32)
            for v in range(W // L):
                idx_v[t, pl.ds(v * L, L)] = rep
        rbufs = (r0, r1)
        gathers = [
            pltpu.async_copy(row_hbm.at[idx_v.at[t]], rbufs[t], (g0, g1)[t])
            for t in range(RPW)
        ]
        writes = []
        for t in range(RPW):
            gathers[t].wait()
            row0 = (base_i + t) * W
            writes.append(
                pltpu.async_copy(
                    rbufs[t], out_hbm.at[pl.ds(row0, W), pl.ds(0, D2)], wsem
                )
            )
        col_read.wait()
        for t in range(RPW):
            row0 = (base_i + t) * W
            writes.append(
                pltpu.async_copy(
                    col_v, out_hbm.at[pl.ds(row0, W), pl.ds(D2, D2)], wsem
                )
            )
        for wcp in writes:
            wcp.wait()

    return dpos_kernel(row_embed, col_embed)


# speedup vs baseline: 1.0772x; 1.0772x over previous
import functools

import jax
import jax.numpy as jnp
from jax import lax
from jax.experimental import pallas as pl
from jax.experimental.pallas import tpu as pltpu
from jax.experimental.pallas import tpu_sc as plsc


def kernel(row_embed, col_embed):
    H, D2 = row_embed.shape
    W = col_embed.shape[0]

    NW = 32
    RPW = H // NW
    L = 16
    NV = D2 // L
    BR = 32

    mesh = plsc.VectorSubcoreMesh(core_axis_name="c", subcore_axis_name="s")

    @functools.partial(
        pl.kernel,
        mesh=mesh,
        out_type=jax.ShapeDtypeStruct((H * W, 2 * D2), jnp.float32),
        scratch_types=[
            pltpu.VMEM((W, D2), jnp.float32),
            pltpu.VMEM((RPW, D2), jnp.float32),
            pltpu.VMEM((BR, D2), jnp.float32),
            pltpu.VMEM((BR, D2), jnp.float32),
            pltpu.SemaphoreType.DMA,
            pltpu.SemaphoreType.DMA,
        ],
    )
    def dpos_kernel(row_hbm, col_hbm, out_hbm, col_v, myrows_v, b0, b1, rsem, wsem):
        wid = lax.axis_index("s") * 2 + lax.axis_index("c")
        base_i = wid * RPW
        pltpu.sync_copy(row_hbm.at[pl.ds(base_i, RPW)], myrows_v)
        col_read = pltpu.async_copy(col_hbm, col_v, rsem)
        writes = []
        bufs = (b0, b1)
        col_pending = True
        for t in range(RPW):
            bcast = bufs[t]
            vals = [myrows_v[t, pl.ds(v * L, L)] for v in range(NV)]
            for r in range(BR):
                for v in range(NV):
                    bcast[r, pl.ds(v * L, L)] = vals[v]
            row0 = (base_i + t) * W
            for q in range(W // BR):
                writes.append(
                    pltpu.async_copy(
                        bcast,
                        out_hbm.at[pl.ds(row0 + q * BR, BR), pl.ds(0, D2)],
                        wsem,
                    )
                )
            if col_pending:
                col_read.wait()
                col_pending = False
                for u in range(RPW):
                    writes.append(
                        pltpu.async_copy(
                            col_v,
                            out_hbm.at[pl.ds((base_i + u) * W, W), pl.ds(D2, D2)],
                            wsem,
                        )
                    )
        for wcp in writes:
            wcp.wait()

    return dpos_kernel(row_embed, col_embed)


# speedup vs baseline: 1.2131x; 1.1261x over previous
import functools

import jax
import jax.numpy as jnp
from jax import lax
from jax.experimental import pallas as pl
from jax.experimental.pallas import tpu as pltpu
from jax.experimental.pallas import tpu_sc as plsc


def kernel(row_embed, col_embed):
    H, D2 = row_embed.shape
    W = col_embed.shape[0]

    NW = 32
    RPW = H // NW
    L = 16
    NV = D2 // L
    BR = 8

    mesh = plsc.VectorSubcoreMesh(core_axis_name="c", subcore_axis_name="s")

    @functools.partial(
        pl.kernel,
        mesh=mesh,
        out_type=jax.ShapeDtypeStruct((H * W, 2 * D2), jnp.float32),
        scratch_types=[
            pltpu.VMEM((W, D2), jnp.float32),
            pltpu.VMEM((RPW, D2), jnp.float32),
            pltpu.VMEM((BR, D2), jnp.float32),
            pltpu.VMEM((BR, D2), jnp.float32),
            pltpu.SemaphoreType.DMA,
            pltpu.SemaphoreType.DMA,
        ],
    )
    def dpos_kernel(row_hbm, col_hbm, out_hbm, col_v, myrows_v, b0, b1, rsem, wsem):
        wid = lax.axis_index("s") * 2 + lax.axis_index("c")
        base_i = wid * RPW
        col_read = pltpu.async_copy(col_hbm, col_v, rsem)
        pltpu.sync_copy(row_hbm.at[pl.ds(base_i, RPW)], myrows_v)
        writes = []
        bufs = (b0, b1)
        for t in range(RPW):
            bcast = bufs[t]
            vals = [myrows_v[t, pl.ds(v * L, L)] for v in range(NV)]
            for r in range(BR):
                for v in range(NV):
                    bcast[r, pl.ds(v * L, L)] = vals[v]
            row0 = (base_i + t) * W
            for q in range(W // BR):
                writes.append(
                    pltpu.async_copy(
                        bcast,
                        out_hbm.at[pl.ds(row0 + q * BR, BR), pl.ds(0, D2)],
                        wsem,
                    )
                )
        col_read.wait()
        for t in range(RPW):
            row0 = (base_i + t) * W
            writes.append(
                pltpu.async_copy(
                    col_v,
                    out_hbm.at[pl.ds(row0, W), pl.ds(D2, D2)],
                    wsem,
                )
            )
        for wcp in writes:
            wcp.wait()

    return dpos_kernel(row_embed, col_embed)
